# baseline (speedup 1.0000x reference)
import functools

import jax
import jax.numpy as jnp
from jax import lax
from jax.experimental import pallas as pl
from jax.experimental.pallas import tpu as pltpu
from jax.experimental.pallas import tpu_sc as plsc

_VBLK = 2048
_NQ = 4
_SLAB = 8


def _gather_slabs(idx8, table3):
    info = plsc.get_sparse_core_info()
    nc, ns = info.num_cores, info.num_subcores
    nw = nc * ns
    b = idx8.shape[0]
    d2 = table3.shape[1]
    b_per_w = b // nw
    mesh = plsc.VectorSubcoreMesh(core_axis_name="c", subcore_axis_name="s")

    @functools.partial(
        pl.kernel,
        mesh=mesh,
        out_type=jax.ShapeDtypeStruct((b, d2), table3.dtype),
        scratch_types=[
            pltpu.VMEM((b_per_w,), jnp.int32),
            pltpu.VMEM((b_per_w, d2), table3.dtype),
            pltpu.SemaphoreType.DMA,
        ],
    )
    def gather_k(idx_hbm, table_hbm, out_hbm, idx_v, rows_v, sem):
        wid = lax.axis_index("s") * nc + lax.axis_index("c")
        base = wid * b_per_w
        pltpu.sync_copy(idx_hbm.at[pl.ds(base, b_per_w)], idx_v)
        pltpu.async_copy(table_hbm.at[idx_v], rows_v, sem).wait()
        pltpu.sync_copy(rows_v, out_hbm.at[pl.ds(base, b_per_w)])

    return gather_k(idx8, table3)


def _make_mm_body(b, d, nsteps):
    def copies(buf_ref, out_hbm, sem_ref, slot, step, nrows):
        chunk = nrows // _NQ
        cps = []
        for q in range(_NQ):
            cps.append(
                pltpu.make_async_copy(
                    buf_ref.at[slot, pl.ds(q * chunk, chunk)],
                    out_hbm.at[pl.ds(step * _VBLK + q * chunk, chunk)],
                    sem_ref.at[slot, q],
                )
            )
        return cps

    def body(w_ref, slab_ref, sub_ref, out_hbm, vc_ref, buf_ref, sem_ref):
        i = pl.program_id(0)
        vocab = out_hbm.shape[0]
        last_rows = vocab - (nsteps - 1) * _VBLK

        @pl.when(i == 0)
        def _():
            vc_ref[...] = jnp.where(
                sub_ref[...] != 0, slab_ref[:, d:], slab_ref[:, :d]
            )

        @pl.when(i >= 2)
        def _():
            for slot in range(2):
                @pl.when(lax.rem(i, 2) == slot)
                def _():
                    for cp in copies(buf_ref, out_hbm, sem_ref, slot, i - 2, _VBLK):
                        cp.wait()

        acc = lax.dot_general(
            w_ref[...],
            vc_ref[...],
            dimension_numbers=(((0,), (1,)), ((), ())),
            preferred_element_type=jnp.float32,
        )
        for slot in range(2):
            @pl.when(lax.rem(i, 2) == slot)
            def _():
                buf_ref[slot] = acc
                @pl.when(i < nsteps - 1)
                def _():
                    for cp in copies(buf_ref, out_hbm, sem_ref, slot, i, _VBLK):
                        cp.start()

        @pl.when(i == nsteps - 1)
        def _():
            lslot = (nsteps - 1) % 2
            for cp in copies(buf_ref, out_hbm, sem_ref, lslot, nsteps - 1, last_rows):
                cp.start()
            if nsteps >= 2:
                pslot = (nsteps - 2) % 2
                for cp in copies(buf_ref, out_hbm, sem_ref, pslot, nsteps - 2, _VBLK):
                    cp.wait()
            for cp in copies(buf_ref, out_hbm, sem_ref, lslot, nsteps - 1, last_rows):
                cp.wait()

    return body


def kernel(center_word_index, W_in, W_out):
    idx = center_word_index.astype(jnp.int32)
    vocab, d = W_in.shape
    b = idx.shape[0]

    table2 = W_in.reshape(vocab // 2, 2 * d)
    wide = _gather_slabs(idx >> 1, table2)
    sub = (idx & 1).reshape(b, 1)

    w_t = W_out.T
    nsteps = pl.cdiv(vocab, _VBLK)

    scores_t = pl.pallas_call(
        _make_mm_body(b, d, nsteps),
        grid=(nsteps,),
        in_specs=[
            pl.BlockSpec((d, _VBLK), lambda i: (0, i)),
            pl.BlockSpec((b, 2 * d), lambda i: (0, 0)),
            pl.BlockSpec((b, 1), lambda i: (0, 0)),
        ],
        out_specs=pl.BlockSpec(memory_space=pl.ANY),
        out_shape=jax.ShapeDtypeStruct((vocab, b), jnp.float32),
        scratch_shapes=[
            pltpu.VMEM((b, d), jnp.float32),
            pltpu.VMEM((2, _VBLK, b), jnp.float32),
            pltpu.SemaphoreType.DMA((2, _NQ)),
        ],
    )(w_t, wide, sub)

    return scores_t.T


# speedup vs baseline: 1.1482x; 1.1482x over previous
import functools

import jax
import jax.numpy as jnp
from jax import lax
from jax.experimental import pallas as pl
from jax.experimental.pallas import tpu as pltpu
from jax.experimental.pallas import tpu_sc as plsc

_VBLK = 2048
_NQ = 4
_SLAB = 8


def _gather_slabs(idx8, table3):
    info = plsc.get_sparse_core_info()
    nc, ns = info.num_cores, info.num_subcores
    nw = nc * ns
    b = idx8.shape[0]
    d2 = table3.shape[1]
    b_per_w = b // nw
    mesh = plsc.VectorSubcoreMesh(core_axis_name="c", subcore_axis_name="s")

    @functools.partial(
        pl.kernel,
        mesh=mesh,
        out_type=jax.ShapeDtypeStruct((b, d2), table3.dtype),
        scratch_types=[
            pltpu.VMEM((b_per_w,), jnp.int32),
            pltpu.VMEM((b_per_w, d2), table3.dtype),
            pltpu.SemaphoreType.DMA,
        ],
    )
    def gather_k(idx_hbm, table_hbm, out_hbm, idx_v, rows_v, sem):
        wid = lax.axis_index("s") * nc + lax.axis_index("c")
        base = wid * b_per_w
        pltpu.sync_copy(idx_hbm.at[pl.ds(base, b_per_w)], idx_v)
        pltpu.async_copy(table_hbm.at[idx_v], rows_v, sem).wait()
        pltpu.sync_copy(rows_v, out_hbm.at[pl.ds(base, b_per_w)])

    return gather_k(idx8, table3)


_TBLK = 2048
_KSPLIT = 25 * _TBLK


def _pair_table_body(lo_ref, hi_ref, out_ref):
    d = lo_ref.shape[0]
    out_ref[:, :d] = lo_ref[...].T
    out_ref[:, d:] = hi_ref[...].T


def _make_pair_table(w_t, vocab, d):
    nblk = _KSPLIT // _TBLK
    last_blk = (vocab - 1) // _TBLK
    return pl.pallas_call(
        _pair_table_body,
        grid=(nblk,),
        in_specs=[
            pl.BlockSpec((d, _TBLK), lambda i: (0, i)),
            pl.BlockSpec((d, _TBLK), lambda i: (0, jnp.minimum(i + nblk, last_blk))),
        ],
        out_specs=pl.BlockSpec((_TBLK, 2 * d), lambda i: (i, 0)),
        out_shape=jax.ShapeDtypeStruct((_KSPLIT, 2 * d), jnp.float32),
    )(w_t, w_t)


def _make_mm_body(b, d, nsteps):
    def copies(buf_ref, out_hbm, sem_ref, slot, step, nrows):
        chunk = nrows // _NQ
        cps = []
        for q in range(_NQ):
            cps.append(
                pltpu.make_async_copy(
                    buf_ref.at[slot, pl.ds(q * chunk, chunk)],
                    out_hbm.at[pl.ds(step * _VBLK + q * chunk, chunk)],
                    sem_ref.at[slot, q],
                )
            )
        return cps

    def body(w_ref, slab_ref, sub_ref, out_hbm, vc_ref, buf_ref, sem_ref):
        i = pl.program_id(0)
        vocab = out_hbm.shape[0]
        last_rows = vocab - (nsteps - 1) * _VBLK

        @pl.when(i == 0)
        def _():
            vc_ref[...] = jnp.where(
                sub_ref[...] != 0, slab_ref[:, d:], slab_ref[:, :d]
            )

        @pl.when(i >= 2)
        def _():
            for slot in range(2):
                @pl.when(lax.rem(i, 2) == slot)
                def _():
                    for cp in copies(buf_ref, out_hbm, sem_ref, slot, i - 2, _VBLK):
                        cp.wait()

        acc = lax.dot_general(
            w_ref[...],
            vc_ref[...],
            dimension_numbers=(((0,), (1,)), ((), ())),
            preferred_element_type=jnp.float32,
        )
        for slot in range(2):
            @pl.when(lax.rem(i, 2) == slot)
            def _():
                buf_ref[slot] = acc
                @pl.when(i < nsteps - 1)
                def _():
                    for cp in copies(buf_ref, out_hbm, sem_ref, slot, i, _VBLK):
                        cp.start()

        @pl.when(i == nsteps - 1)
        def _():
            lslot = (nsteps - 1) % 2
            for cp in copies(buf_ref, out_hbm, sem_ref, lslot, nsteps - 1, last_rows):
                cp.start()
            if nsteps >= 2:
                pslot = (nsteps - 2) % 2
                for cp in copies(buf_ref, out_hbm, sem_ref, pslot, nsteps - 2, _VBLK):
                    cp.wait()
            for cp in copies(buf_ref, out_hbm, sem_ref, lslot, nsteps - 1, last_rows):
                cp.wait()

    return body


def kernel(center_word_index, W_in, W_out):
    idx = center_word_index.astype(jnp.int32)
    vocab, d = W_in.shape
    b = idx.shape[0]

    table2 = _make_pair_table(W_in.T, vocab, d)
    idxm = jnp.where(idx >= _KSPLIT, idx - _KSPLIT, idx)
    wide = _gather_slabs(idxm, table2)
    sub = (idx >= _KSPLIT).astype(jnp.int32).reshape(b, 1)

    w_t = W_out.T
    nsteps = pl.cdiv(vocab, _VBLK)

    scores_t = pl.pallas_call(
        _make_mm_body(b, d, nsteps),
        grid=(nsteps,),
        in_specs=[
            pl.BlockSpec((d, _VBLK), lambda i: (0, i)),
            pl.BlockSpec((b, 2 * d), lambda i: (0, 0)),
            pl.BlockSpec((b, 1), lambda i: (0, 0)),
        ],
        out_specs=pl.BlockSpec(memory_space=pl.ANY),
        out_shape=jax.ShapeDtypeStruct((vocab, b), jnp.float32),
        scratch_shapes=[
            pltpu.VMEM((b, d), jnp.float32),
            pltpu.VMEM((2, _VBLK, b), jnp.float32),
            pltpu.SemaphoreType.DMA((2, _NQ)),
        ],
    )(w_t, wide, sub)

    return scores_t.T


# speedup vs baseline: 1.1944x; 1.0402x over previous
import functools

import jax
import jax.numpy as jnp
from jax import lax
from jax.experimental import pallas as pl
from jax.experimental.pallas import tpu as pltpu
from jax.experimental.pallas import tpu_sc as plsc

_VBLK = 2048
_NQ = 4
_SLAB = 8


def _gather_slabs(idx8, table3):
    info = plsc.get_sparse_core_info()
    nc, ns = info.num_cores, info.num_subcores
    nw = nc * ns
    b = idx8.shape[0]
    d2 = table3.shape[1]
    b_per_w = b // nw
    mesh = plsc.VectorSubcoreMesh(core_axis_name="c", subcore_axis_name="s")

    @functools.partial(
        pl.kernel,
        mesh=mesh,
        out_type=jax.ShapeDtypeStruct((b, d2), table3.dtype),
        scratch_types=[
            pltpu.VMEM((b_per_w,), jnp.int32),
            pltpu.VMEM((b_per_w, d2), table3.dtype),
            pltpu.SemaphoreType.DMA,
        ],
    )
    def gather_k(idx_hbm, table_hbm, out_hbm, idx_v, rows_v, sem):
        wid = lax.axis_index("s") * nc + lax.axis_index("c")
        base = wid * b_per_w
        pltpu.sync_copy(idx_hbm.at[pl.ds(base, b_per_w)], idx_v)
        pltpu.async_copy(table_hbm.at[idx_v], rows_v, sem).wait()
        pltpu.sync_copy(rows_v, out_hbm.at[pl.ds(base, b_per_w)])

    return gather_k(idx8, table3)


_TBLK = 5120
_KSPLIT = 10 * _TBLK


def _pair_table_body(lo_ref, hi_ref, out_ref):
    d = lo_ref.shape[0]
    out_ref[:, :d] = lo_ref[...].T
    out_ref[:, d:] = hi_ref[...].T


def _make_pair_table(w_t, vocab, d):
    nblk = _KSPLIT // _TBLK
    last_blk = (vocab - 1) // _TBLK
    return pl.pallas_call(
        _pair_table_body,
        grid=(nblk,),
        in_specs=[
            pl.BlockSpec((d, _TBLK), lambda i: (0, i)),
            pl.BlockSpec((d, _TBLK), lambda i: (0, jnp.minimum(i + nblk, last_blk))),
        ],
        out_specs=pl.BlockSpec((_TBLK, 2 * d), lambda i: (i, 0)),
        out_shape=jax.ShapeDtypeStruct((_KSPLIT, 2 * d), jnp.float32),
    )(w_t, w_t)


def _make_mm_body(b, d, nsteps):
    def copies(buf_ref, out_hbm, sem_ref, slot, step, nrows):
        chunk = nrows // _NQ
        cps = []
        for q in range(_NQ):
            cps.append(
                pltpu.make_async_copy(
                    buf_ref.at[slot, pl.ds(q * chunk, chunk)],
                    out_hbm.at[pl.ds(step * _VBLK + q * chunk, chunk)],
                    sem_ref.at[slot, q],
                )
            )
        return cps

    def body(w_ref, slab_ref, sub_ref, out_hbm, vc_ref, buf_ref, sem_ref):
        i = pl.program_id(0)
        vocab = out_hbm.shape[0]
        last_rows = vocab - (nsteps - 1) * _VBLK

        @pl.when(i == 0)
        def _():
            vc_ref[...] = jnp.where(
                sub_ref[...] != 0, slab_ref[:, d:], slab_ref[:, :d]
            )

        @pl.when(i >= 2)
        def _():
            for slot in range(2):
                @pl.when(lax.rem(i, 2) == slot)
                def _():
                    for cp in copies(buf_ref, out_hbm, sem_ref, slot, i - 2, _VBLK):
                        cp.wait()

        acc = lax.dot_general(
            w_ref[...],
            vc_ref[...],
            dimension_numbers=(((0,), (1,)), ((), ())),
            preferred_element_type=jnp.float32,
        )
        for slot in range(2):
            @pl.when(lax.rem(i, 2) == slot)
            def _():
                buf_ref[slot] = acc
                @pl.when(i < nsteps - 1)
                def _():
                    for cp in copies(buf_ref, out_hbm, sem_ref, slot, i, _VBLK):
                        cp.start()

        @pl.when(i == nsteps - 1)
        def _():
            lslot = (nsteps - 1) % 2
            for cp in copies(buf_ref, out_hbm, sem_ref, lslot, nsteps - 1, last_rows):
                cp.start()
            if nsteps >= 2:
                pslot = (nsteps - 2) % 2
                for cp in copies(buf_ref, out_hbm, sem_ref, pslot, nsteps - 2, _VBLK):
                    cp.wait()
            for cp in copies(buf_ref, out_hbm, sem_ref, lslot, nsteps - 1, last_rows):
                cp.wait()

    return body


def kernel(center_word_index, W_in, W_out):
    idx = center_word_index.astype(jnp.int32)
    vocab, d = W_in.shape
    b = idx.shape[0]

    table2 = _make_pair_table(W_in.T, vocab, d)
    idxm = jnp.where(idx >= _KSPLIT, idx - _KSPLIT, idx)
    wide = _gather_slabs(idxm, table2)
    sub = (idx >= _KSPLIT).astype(jnp.int32).reshape(b, 1)

    w_t = W_out.T
    nsteps = pl.cdiv(vocab, _VBLK)

    scores_t = pl.pallas_call(
        _make_mm_body(b, d, nsteps),
        grid=(nsteps,),
        in_specs=[
            pl.BlockSpec((d, _VBLK), lambda i: (0, i)),
            pl.BlockSpec((b, 2 * d), lambda i: (0, 0)),
            pl.BlockSpec((b, 1), lambda i: (0, 0)),
        ],
        out_specs=pl.BlockSpec(memory_space=pl.ANY),
        out_shape=jax.ShapeDtypeStruct((vocab, b), jnp.float32),
        scratch_shapes=[
            pltpu.VMEM((b, d), jnp.float32),
            pltpu.VMEM((2, _VBLK, b), jnp.float32),
            pltpu.SemaphoreType.DMA((2, _NQ)),
        ],
    )(w_t, wide, sub)

    return scores_t.T


# speedup vs baseline: 1.1998x; 1.0046x over previous
import functools

import jax
import jax.numpy as jnp
from jax import lax
from jax.experimental import pallas as pl
from jax.experimental.pallas import tpu as pltpu
from jax.experimental.pallas import tpu_sc as plsc

_VBLK = 4096
_NQ = 4
_SLAB = 8


def _gather_slabs(idx8, table3):
    info = plsc.get_sparse_core_info()
    nc, ns = info.num_cores, info.num_subcores
    nw = nc * ns
    b = idx8.shape[0]
    d2 = table3.shape[1]
    b_per_w = b // nw
    mesh = plsc.VectorSubcoreMesh(core_axis_name="c", subcore_axis_name="s")

    @functools.partial(
        pl.kernel,
        mesh=mesh,
        out_type=jax.ShapeDtypeStruct((b, d2), table3.dtype),
        scratch_types=[
            pltpu.VMEM((b_per_w,), jnp.int32),
            pltpu.VMEM((b_per_w, d2), table3.dtype),
            pltpu.SemaphoreType.DMA,
        ],
    )
    def gather_k(idx_hbm, table_hbm, out_hbm, idx_v, rows_v, sem):
        wid = lax.axis_index("s") * nc + lax.axis_index("c")
        base = wid * b_per_w
        pltpu.sync_copy(idx_hbm.at[pl.ds(base, b_per_w)], idx_v)
        pltpu.async_copy(table_hbm.at[idx_v], rows_v, sem).wait()
        pltpu.sync_copy(rows_v, out_hbm.at[pl.ds(base, b_per_w)])

    return gather_k(idx8, table3)


_TBLK = 5120
_KSPLIT = 10 * _TBLK


def _pair_table_body(lo_ref, hi_ref, out_ref):
    d = lo_ref.shape[0]
    out_ref[:, :d] = lo_ref[...].T
    out_ref[:, d:] = hi_ref[...].T


def _make_pair_table(w_t, vocab, d):
    nblk = _KSPLIT // _TBLK
    last_blk = (vocab - 1) // _TBLK
    return pl.pallas_call(
        _pair_table_body,
        grid=(nblk,),
        in_specs=[
            pl.BlockSpec((d, _TBLK), lambda i: (0, i)),
            pl.BlockSpec((d, _TBLK), lambda i: (0, jnp.minimum(i + nblk, last_blk))),
        ],
        out_specs=pl.BlockSpec((_TBLK, 2 * d), lambda i: (i, 0)),
        out_shape=jax.ShapeDtypeStruct((_KSPLIT, 2 * d), jnp.float32),
    )(w_t, w_t)


def _make_mm_body(b, d, nsteps):
    def copies(buf_ref, out_hbm, sem_ref, slot, step, nrows):
        chunk = nrows // _NQ
        cps = []
        for q in range(_NQ):
            cps.append(
                pltpu.make_async_copy(
                    buf_ref.at[slot, pl.ds(q * chunk, chunk)],
                    out_hbm.at[pl.ds(step * _VBLK + q * chunk, chunk)],
                    sem_ref.at[slot, q],
                )
            )
        return cps

    def body(w_ref, slab_ref, sub_ref, out_hbm, vc_ref, buf_ref, sem_ref):
        i = pl.program_id(0)
        vocab = out_hbm.shape[0]
        last_rows = vocab - (nsteps - 1) * _VBLK

        @pl.when(i == 0)
        def _():
            vc_ref[...] = jnp.where(
                sub_ref[...] != 0, slab_ref[:, d:], slab_ref[:, :d]
            )

        @pl.when(i >= 2)
        def _():
            for slot in range(2):
                @pl.when(lax.rem(i, 2) == slot)
                def _():
                    for cp in copies(buf_ref, out_hbm, sem_ref, slot, i - 2, _VBLK):
                        cp.wait()

        acc = lax.dot_general(
            w_ref[...],
            vc_ref[...],
            dimension_numbers=(((0,), (1,)), ((), ())),
            preferred_element_type=jnp.float32,
        )
        for slot in range(2):
            @pl.when(lax.rem(i, 2) == slot)
            def _():
                buf_ref[slot] = acc
                @pl.when(i < nsteps - 1)
                def _():
                    for cp in copies(buf_ref, out_hbm, sem_ref, slot, i, _VBLK):
                        cp.start()

        @pl.when(i == nsteps - 1)
        def _():
            lslot = (nsteps - 1) % 2
            for cp in copies(buf_ref, out_hbm, sem_ref, lslot, nsteps - 1, last_rows):
                cp.start()
            if nsteps >= 2:
                pslot = (nsteps - 2) % 2
                for cp in copies(buf_ref, out_hbm, sem_ref, pslot, nsteps - 2, _VBLK):
                    cp.wait()
            for cp in copies(buf_ref, out_hbm, sem_ref, lslot, nsteps - 1, last_rows):
                cp.wait()

    return body


def kernel(center_word_index, W_in, W_out):
    idx = center_word_index.astype(jnp.int32)
    vocab, d = W_in.shape
    b = idx.shape[0]

    table2 = _make_pair_table(W_in.T, vocab, d)
    idxm = jnp.where(idx >= _KSPLIT, idx - _KSPLIT, idx)
    wide = _gather_slabs(idxm, table2)
    sub = (idx >= _KSPLIT).astype(jnp.int32).reshape(b, 1)

    w_t = W_out.T
    nsteps = pl.cdiv(vocab, _VBLK)

    scores_t = pl.pallas_call(
        _make_mm_body(b, d, nsteps),
        grid=(nsteps,),
        in_specs=[
            pl.BlockSpec((d, _VBLK), lambda i: (0, i)),
            pl.BlockSpec((b, 2 * d), lambda i: (0, 0)),
            pl.BlockSpec((b, 1), lambda i: (0, 0)),
        ],
        out_specs=pl.BlockSpec(memory_space=pl.ANY),
        out_shape=jax.ShapeDtypeStruct((vocab, b), jnp.float32),
        scratch_shapes=[
            pltpu.VMEM((b, d), jnp.float32),
            pltpu.VMEM((2, _VBLK, b), jnp.float32),
            pltpu.SemaphoreType.DMA((2, _NQ)),
        ],
    )(w_t, wide, sub)

    return scores_t.T
